# Initial kernel scaffold; baseline (speedup 1.0000x reference)
import jax
import jax.numpy as jnp
from jax.experimental import pallas as pl


def kernel(classification, localization, anchors, annotations):
    raise NotImplementedError("write your pallas kernel here")


# speedup vs baseline: 5.5515x; 5.5515x over previous
import jax
import jax.numpy as jnp
from jax import lax
from jax.experimental import pallas as pl
from jax.experimental.pallas import tpu as pltpu

_ALPHA = 0.25
_B, _A, _K, _M = 8, 20000, 80, 32
_BA = 2048
_NB = (_A + _BA - 1) // _BA


def _body(cls_ref, loc_ref, anch_ref, anno_ref, ocls_ref, oloc_ref, acc_ref):
    b = pl.program_id(0)
    j = pl.program_id(1)

    ax1 = anch_ref[0:1, :]
    ay1 = anch_ref[1:2, :]
    ax2 = anch_ref[2:3, :]
    ay2 = anch_ref[3:4, :]
    area_a = (ax2 - ax1) * (ay2 - ay1)

    best = jnp.full((1, _BA), -1.0, jnp.float32)
    gx1 = jnp.zeros((1, _BA), jnp.float32)
    gy1 = jnp.zeros((1, _BA), jnp.float32)
    gx2 = jnp.zeros((1, _BA), jnp.float32)
    gy2 = jnp.zeros((1, _BA), jnp.float32)
    gcl = jnp.zeros((1, _BA), jnp.float32)
    for m in range(_M):
        bx1 = anno_ref[b, m, 0]
        by1 = anno_ref[b, m, 1]
        bx2 = anno_ref[b, m, 2]
        by2 = anno_ref[b, m, 3]
        bcl = anno_ref[b, m, 4]
        area_b = (bx2 - bx1) * (by2 - by1)
        iw = jnp.maximum(jnp.minimum(ax2, bx2) - jnp.maximum(ax1, bx1), 0.0)
        ih = jnp.maximum(jnp.minimum(ay2, by2) - jnp.maximum(ay1, by1), 0.0)
        inter = iw * ih
        union = jnp.maximum(area_a + (area_b - inter), 1e-8)
        iou = inter / union
        cond = iou > best
        best = jnp.where(cond, iou, best)
        gx1 = jnp.where(cond, bx1, gx1)
        gy1 = jnp.where(cond, by1, gy1)
        gx2 = jnp.where(cond, bx2, gx2)
        gy2 = jnp.where(cond, by2, gy2)
        gcl = jnp.where(cond, bcl, gcl)

    gid = j * _BA + lax.broadcasted_iota(jnp.int32, (1, _BA), 1)
    valid = gid < _A
    pos = best >= 0.5
    posv = jnp.logical_and(pos, valid)
    ign = jnp.logical_or(
        jnp.logical_not(valid),
        jnp.logical_and(jnp.logical_not(pos), best >= 0.4),
    )
    code = jnp.where(ign, -1.0, jnp.where(posv, gcl, -2.0))
    pos_cnt = jnp.sum(posv.astype(jnp.float32))

    p = jnp.clip(cls_ref[0], 1e-4, 1.0 - 1e-4)
    code_t = code.reshape(_BA, 1)
    kio = lax.broadcasted_iota(jnp.int32, (_BA, _K), 1).astype(jnp.float32)
    t2 = code_t == kio
    ig2 = code_t == -1.0
    q = jnp.where(t2, 1.0 - p, p)
    al = jnp.where(t2, _ALPHA, 1.0 - _ALPHA)
    lg = jnp.log(1.0 - q)
    cls_blk = jnp.sum(jnp.where(ig2, 0.0, -(al * q * q * lg)))

    aw = ax2 - ax1
    ah = ay2 - ay1
    acx = ax1 + 0.5 * aw
    acy = ay1 + 0.5 * ah
    gw0 = gx2 - gx1
    gh0 = gy2 - gy1
    gcx = gx1 + 0.5 * gw0
    gcy = gy1 + 0.5 * gh0
    gw = jnp.maximum(gw0, 1.0)
    gh = jnp.maximum(gh0, 1.0)
    dx = (gcx - acx) / aw / 0.1
    dy = (gcy - acy) / ah / 0.1
    dw = jnp.log(gw / aw) / 0.2
    dh = jnp.log(gh / ah) / 0.2
    l4 = loc_ref[0]
    loc_blk = jnp.float32(0.0)
    for i, d in enumerate((dx, dy, dw, dh)):
        diff = jnp.abs(d - l4[i : i + 1, :])
        sl1 = jnp.where(diff <= 1.0 / 9.0, 0.5 * 9.0 * diff * diff, diff - 0.5 / 9.0)
        loc_blk = loc_blk + jnp.sum(jnp.where(posv, sl1, 0.0))

    @pl.when(jnp.logical_and(b == 0, j == 0))
    def _():
        acc_ref[3] = 0.0
        acc_ref[4] = 0.0

    @pl.when(j == 0)
    def _():
        acc_ref[0] = 0.0
        acc_ref[1] = 0.0
        acc_ref[2] = 0.0

    acc_ref[0] = acc_ref[0] + cls_blk
    acc_ref[1] = acc_ref[1] + loc_blk
    acc_ref[2] = acc_ref[2] + pos_cnt

    @pl.when(j == _NB - 1)
    def _():
        pn = acc_ref[2]
        cls_img = acc_ref[0] / jnp.maximum(pn, 1.0)
        loc_img = jnp.where(pn > 0.0, acc_ref[1] / jnp.maximum(pn * 4.0, 1.0), 0.0)
        acc_ref[3] = acc_ref[3] + cls_img
        acc_ref[4] = acc_ref[4] + loc_img

        @pl.when(b == _B - 1)
        def _():
            ocls_ref[0, 0] = acc_ref[3] / _B
            oloc_ref[0, 0] = acc_ref[4] / _B


def kernel(classification, localization, anchors, annotations):
    loc_t = jnp.transpose(localization, (0, 2, 1))
    anch_t = jnp.transpose(anchors)
    ocls, oloc = pl.pallas_call(
        _body,
        grid=(_B, _NB),
        in_specs=[
            pl.BlockSpec((1, _BA, _K), lambda b, j: (b, j, 0)),
            pl.BlockSpec((1, 4, _BA), lambda b, j: (b, 0, j)),
            pl.BlockSpec((4, _BA), lambda b, j: (0, j)),
            pl.BlockSpec(memory_space=pltpu.SMEM),
        ],
        out_specs=[
            pl.BlockSpec(memory_space=pltpu.SMEM),
            pl.BlockSpec(memory_space=pltpu.SMEM),
        ],
        out_shape=[
            jax.ShapeDtypeStruct((1, 1), jnp.float32),
            jax.ShapeDtypeStruct((1, 1), jnp.float32),
        ],
        scratch_shapes=[pltpu.SMEM((8,), jnp.float32)],
    )(classification, loc_t, anch_t, annotations)
    return (ocls.reshape(1), oloc.reshape(1))


# speedup vs baseline: 6.1402x; 1.1060x over previous
import jax
import jax.numpy as jnp
from jax import lax
from jax.experimental import pallas as pl
from jax.experimental.pallas import tpu as pltpu

_ALPHA = 0.25
_B, _A, _K, _M = 8, 20000, 80, 32
_BA = 4096
_NB = (_A + _BA - 1) // _BA


def _body(cls_ref, loc_ref, anch_ref, anno_ref, ocls_ref, oloc_ref, acc_ref):
    b = pl.program_id(0)
    j = pl.program_id(1)

    ax1 = anch_ref[0:1, :]
    ay1 = anch_ref[1:2, :]
    ax2 = anch_ref[2:3, :]
    ay2 = anch_ref[3:4, :]
    area_a = (ax2 - ax1) * (ay2 - ay1)

    best = jnp.full((1, _BA), -1.0, jnp.float32)
    gx1 = jnp.zeros((1, _BA), jnp.float32)
    gy1 = jnp.zeros((1, _BA), jnp.float32)
    gx2 = jnp.zeros((1, _BA), jnp.float32)
    gy2 = jnp.zeros((1, _BA), jnp.float32)
    gcl = jnp.zeros((1, _BA), jnp.float32)
    for m in range(_M):
        bx1 = anno_ref[b, m, 0]
        by1 = anno_ref[b, m, 1]
        bx2 = anno_ref[b, m, 2]
        by2 = anno_ref[b, m, 3]
        bcl = anno_ref[b, m, 4]
        area_b = (bx2 - bx1) * (by2 - by1)
        iw = jnp.maximum(jnp.minimum(ax2, bx2) - jnp.maximum(ax1, bx1), 0.0)
        ih = jnp.maximum(jnp.minimum(ay2, by2) - jnp.maximum(ay1, by1), 0.0)
        inter = iw * ih
        union = jnp.maximum(area_a + (area_b - inter), 1e-8)
        iou = inter / union
        cond = iou > best
        best = jnp.where(cond, iou, best)
        gx1 = jnp.where(cond, bx1, gx1)
        gy1 = jnp.where(cond, by1, gy1)
        gx2 = jnp.where(cond, bx2, gx2)
        gy2 = jnp.where(cond, by2, gy2)
        gcl = jnp.where(cond, bcl, gcl)

    gid = j * _BA + lax.broadcasted_iota(jnp.int32, (1, _BA), 1)
    valid = gid < _A
    pos = best >= 0.5
    posv = jnp.logical_and(pos, valid)
    ign = jnp.logical_or(
        jnp.logical_not(valid),
        jnp.logical_and(jnp.logical_not(pos), best >= 0.4),
    )
    code = jnp.where(ign, -1.0, jnp.where(posv, gcl, -2.0))
    pos_cnt = jnp.sum(posv.astype(jnp.float32))

    p = jnp.clip(cls_ref[0], 1e-4, 1.0 - 1e-4)
    code_t = code.reshape(_BA, 1)
    kio = lax.broadcasted_iota(jnp.int32, (_BA, _K), 1).astype(jnp.float32)
    t2 = code_t == kio
    ig2 = code_t == -1.0
    q = jnp.where(t2, 1.0 - p, p)
    al = jnp.where(t2, _ALPHA, 1.0 - _ALPHA)
    lg = jnp.log(1.0 - q)
    cls_blk = jnp.sum(jnp.where(ig2, 0.0, -(al * q * q * lg)))

    aw = ax2 - ax1
    ah = ay2 - ay1
    acx = ax1 + 0.5 * aw
    acy = ay1 + 0.5 * ah
    gw0 = gx2 - gx1
    gh0 = gy2 - gy1
    gcx = gx1 + 0.5 * gw0
    gcy = gy1 + 0.5 * gh0
    gw = jnp.maximum(gw0, 1.0)
    gh = jnp.maximum(gh0, 1.0)
    dx = (gcx - acx) / aw / 0.1
    dy = (gcy - acy) / ah / 0.1
    dw = jnp.log(gw / aw) / 0.2
    dh = jnp.log(gh / ah) / 0.2
    l4 = loc_ref[0]
    loc_blk = jnp.float32(0.0)
    for i, d in enumerate((dx, dy, dw, dh)):
        diff = jnp.abs(d - l4[i : i + 1, :])
        sl1 = jnp.where(diff <= 1.0 / 9.0, 0.5 * 9.0 * diff * diff, diff - 0.5 / 9.0)
        loc_blk = loc_blk + jnp.sum(jnp.where(posv, sl1, 0.0))

    @pl.when(jnp.logical_and(b == 0, j == 0))
    def _():
        acc_ref[3] = 0.0
        acc_ref[4] = 0.0

    @pl.when(j == 0)
    def _():
        acc_ref[0] = 0.0
        acc_ref[1] = 0.0
        acc_ref[2] = 0.0

    acc_ref[0] = acc_ref[0] + cls_blk
    acc_ref[1] = acc_ref[1] + loc_blk
    acc_ref[2] = acc_ref[2] + pos_cnt

    @pl.when(j == _NB - 1)
    def _():
        pn = acc_ref[2]
        cls_img = acc_ref[0] / jnp.maximum(pn, 1.0)
        loc_img = jnp.where(pn > 0.0, acc_ref[1] / jnp.maximum(pn * 4.0, 1.0), 0.0)
        acc_ref[3] = acc_ref[3] + cls_img
        acc_ref[4] = acc_ref[4] + loc_img

        @pl.when(b == _B - 1)
        def _():
            ocls_ref[0, 0] = acc_ref[3] / _B
            oloc_ref[0, 0] = acc_ref[4] / _B


def kernel(classification, localization, anchors, annotations):
    loc_t = jnp.transpose(localization, (0, 2, 1))
    anch_t = jnp.transpose(anchors)
    ocls, oloc = pl.pallas_call(
        _body,
        grid=(_B, _NB),
        in_specs=[
            pl.BlockSpec((1, _BA, _K), lambda b, j: (b, j, 0)),
            pl.BlockSpec((1, 4, _BA), lambda b, j: (b, 0, j)),
            pl.BlockSpec((4, _BA), lambda b, j: (0, j)),
            pl.BlockSpec(memory_space=pltpu.SMEM),
        ],
        out_specs=[
            pl.BlockSpec(memory_space=pltpu.SMEM),
            pl.BlockSpec(memory_space=pltpu.SMEM),
        ],
        out_shape=[
            jax.ShapeDtypeStruct((1, 1), jnp.float32),
            jax.ShapeDtypeStruct((1, 1), jnp.float32),
        ],
        scratch_shapes=[pltpu.SMEM((8,), jnp.float32)],
    )(classification, loc_t, anch_t, annotations)
    return (ocls.reshape(1), oloc.reshape(1))


# speedup vs baseline: 7.1031x; 1.1568x over previous
import jax
import jax.numpy as jnp
from jax import lax
from jax.experimental import pallas as pl
from jax.experimental.pallas import tpu as pltpu

_ALPHA = 0.25
_B, _A, _K, _M = 8, 20000, 80, 32
_BA = 4096
_NB = (_A + _BA - 1) // _BA


def _body(cls_ref, loc_ref, anch_ref, anno_ref, ocls_ref, oloc_ref, acc_ref):
    b = pl.program_id(0)
    j = pl.program_id(1)

    ax1 = anch_ref[0:1, :]
    ay1 = anch_ref[1:2, :]
    ax2 = anch_ref[2:3, :]
    ay2 = anch_ref[3:4, :]
    area_a = (ax2 - ax1) * (ay2 - ay1)

    best = jnp.full((1, _BA), -1.0, jnp.float32)
    gx1 = jnp.zeros((1, _BA), jnp.float32)
    gy1 = jnp.zeros((1, _BA), jnp.float32)
    gx2 = jnp.zeros((1, _BA), jnp.float32)
    gy2 = jnp.zeros((1, _BA), jnp.float32)
    gcl = jnp.zeros((1, _BA), jnp.float32)
    for m in range(_M):
        bx1 = anno_ref[b, m, 0]
        by1 = anno_ref[b, m, 1]
        bx2 = anno_ref[b, m, 2]
        by2 = anno_ref[b, m, 3]
        bcl = anno_ref[b, m, 4]
        area_b = (bx2 - bx1) * (by2 - by1)
        iw = jnp.maximum(jnp.minimum(ax2, bx2) - jnp.maximum(ax1, bx1), 0.0)
        ih = jnp.maximum(jnp.minimum(ay2, by2) - jnp.maximum(ay1, by1), 0.0)
        inter = iw * ih
        union = area_a + (area_b - inter)
        iou = inter / union
        cond = iou > best
        best = jnp.where(cond, iou, best)
        gx1 = jnp.where(cond, bx1, gx1)
        gy1 = jnp.where(cond, by1, gy1)
        gx2 = jnp.where(cond, bx2, gx2)
        gy2 = jnp.where(cond, by2, gy2)
        gcl = jnp.where(cond, bcl, gcl)

    gid = j * _BA + lax.broadcasted_iota(jnp.int32, (1, _BA), 1)
    valid = gid < _A
    pos = best >= 0.5
    posv = jnp.logical_and(pos, valid)
    ign = jnp.logical_or(
        jnp.logical_not(valid),
        jnp.logical_and(jnp.logical_not(pos), best >= 0.4),
    )
    code = jnp.where(ign, -1.0, jnp.where(posv, gcl, -2.0))

    p = jnp.clip(jnp.transpose(cls_ref[0], (1, 0)), 1e-4, 1.0 - 1e-4)
    kio = lax.broadcasted_iota(jnp.int32, (_K, _BA), 0).astype(jnp.float32)
    t2 = code == kio
    ig2 = code == -1.0
    q = jnp.where(t2, 1.0 - p, p)
    al = jnp.where(t2, -_ALPHA, _ALPHA - 1.0)
    lg = jnp.log(1.0 - q)
    elem = jnp.where(ig2, 0.0, al * q * q * lg)

    aw = ax2 - ax1
    ah = ay2 - ay1
    acx = ax1 + 0.5 * aw
    acy = ay1 + 0.5 * ah
    gw0 = gx2 - gx1
    gh0 = gy2 - gy1
    gcx = gx1 + 0.5 * gw0
    gcy = gy1 + 0.5 * gh0
    gw = jnp.maximum(gw0, 1.0)
    gh = jnp.maximum(gh0, 1.0)
    dx = (gcx - acx) / aw / 0.1
    dy = (gcy - acy) / ah / 0.1
    dw = jnp.log(gw / aw) / 0.2
    dh = jnp.log(gh / ah) / 0.2
    l4 = loc_ref[0]
    loc_vec = jnp.zeros((1, _BA), jnp.float32)
    for i, d in enumerate((dx, dy, dw, dh)):
        diff = jnp.abs(d - l4[i : i + 1, :])
        sl1 = jnp.where(diff <= 1.0 / 9.0, 0.5 * 9.0 * diff * diff, diff - 0.5 / 9.0)
        loc_vec = loc_vec + jnp.where(posv, sl1, 0.0)
    loc_blk = jnp.sum(loc_vec)
    pos_cnt = jnp.sum(posv.astype(jnp.float32))
    cls_blk = jnp.sum(elem)

    @pl.when(jnp.logical_and(b == 0, j == 0))
    def _():
        acc_ref[3] = 0.0
        acc_ref[4] = 0.0

    @pl.when(j == 0)
    def _():
        acc_ref[0] = 0.0
        acc_ref[1] = 0.0
        acc_ref[2] = 0.0

    acc_ref[0] = acc_ref[0] + cls_blk
    acc_ref[1] = acc_ref[1] + loc_blk
    acc_ref[2] = acc_ref[2] + pos_cnt

    @pl.when(j == _NB - 1)
    def _():
        pn = acc_ref[2]
        cls_img = acc_ref[0] / jnp.maximum(pn, 1.0)
        loc_img = jnp.where(pn > 0.0, acc_ref[1] / jnp.maximum(pn * 4.0, 1.0), 0.0)
        acc_ref[3] = acc_ref[3] + cls_img
        acc_ref[4] = acc_ref[4] + loc_img

        @pl.when(b == _B - 1)
        def _():
            ocls_ref[0, 0] = acc_ref[3] / _B
            oloc_ref[0, 0] = acc_ref[4] / _B


def kernel(classification, localization, anchors, annotations):
    loc_t = jnp.transpose(localization, (0, 2, 1))
    anch_t = jnp.transpose(anchors)
    ocls, oloc = pl.pallas_call(
        _body,
        grid=(_B, _NB),
        in_specs=[
            pl.BlockSpec((1, _BA, _K), lambda b, j: (b, j, 0)),
            pl.BlockSpec((1, 4, _BA), lambda b, j: (b, 0, j)),
            pl.BlockSpec((4, _BA), lambda b, j: (0, j)),
            pl.BlockSpec(memory_space=pltpu.SMEM),
        ],
        out_specs=[
            pl.BlockSpec(memory_space=pltpu.SMEM),
            pl.BlockSpec(memory_space=pltpu.SMEM),
        ],
        out_shape=[
            jax.ShapeDtypeStruct((1, 1), jnp.float32),
            jax.ShapeDtypeStruct((1, 1), jnp.float32),
        ],
        scratch_shapes=[pltpu.SMEM((8,), jnp.float32)],
    )(classification, loc_t, anch_t, annotations)
    return (ocls.reshape(1), oloc.reshape(1))


# speedup vs baseline: 7.3032x; 1.0282x over previous
import jax
import jax.numpy as jnp
from jax import lax
from jax.experimental import pallas as pl
from jax.experimental.pallas import tpu as pltpu

_ALPHA = 0.25
_B, _A, _K, _M = 8, 20000, 80, 32
_BA = 4096
_NB = (_A + _BA - 1) // _BA


def _body(cls_ref, loc_ref, anch_ref, anno_ref, ocls_ref, oloc_ref, acc_ref):
    b = pl.program_id(0)
    j = pl.program_id(1)

    ax1 = anch_ref[0:1, :]
    ay1 = anch_ref[1:2, :]
    ax2 = anch_ref[2:3, :]
    ay2 = anch_ref[3:4, :]
    area_a = (ax2 - ax1) * (ay2 - ay1)

    best = jnp.full((1, _BA), -1.0, jnp.float32)
    gx1 = jnp.zeros((1, _BA), jnp.float32)
    gy1 = jnp.zeros((1, _BA), jnp.float32)
    gx2 = jnp.zeros((1, _BA), jnp.float32)
    gy2 = jnp.zeros((1, _BA), jnp.float32)
    gcl = jnp.zeros((1, _BA), jnp.float32)
    for m in range(_M):
        bx1 = anno_ref[b, m, 0]
        by1 = anno_ref[b, m, 1]
        bx2 = anno_ref[b, m, 2]
        by2 = anno_ref[b, m, 3]
        bcl = anno_ref[b, m, 4]
        area_b = (bx2 - bx1) * (by2 - by1)
        iw = jnp.maximum(jnp.minimum(ax2, bx2) - jnp.maximum(ax1, bx1), 0.0)
        ih = jnp.maximum(jnp.minimum(ay2, by2) - jnp.maximum(ay1, by1), 0.0)
        inter = iw * ih
        union = area_a + (area_b - inter)
        iou = inter / union
        cond = iou > best
        best = jnp.where(cond, iou, best)
        gx1 = jnp.where(cond, bx1, gx1)
        gy1 = jnp.where(cond, by1, gy1)
        gx2 = jnp.where(cond, bx2, gx2)
        gy2 = jnp.where(cond, by2, gy2)
        gcl = jnp.where(cond, bcl, gcl)

    gid = j * _BA + lax.broadcasted_iota(jnp.int32, (1, _BA), 1)
    valid = gid < _A
    pos = best >= 0.5
    posv = jnp.logical_and(pos, valid)
    ign = jnp.logical_or(
        jnp.logical_not(valid),
        jnp.logical_and(jnp.logical_not(pos), best >= 0.4),
    )
    code = jnp.where(ign, -1.0, jnp.where(posv, gcl, -2.0))

    p = jnp.clip(jnp.transpose(cls_ref[0], (1, 0)), 1e-4, 1.0 - 1e-4)
    kio = lax.broadcasted_iota(jnp.int32, (_K, _BA), 0).astype(jnp.float32)
    t2 = code == kio
    ig2 = code == -1.0
    q = jnp.where(t2, 1.0 - p, p)
    al = jnp.where(t2, -_ALPHA, _ALPHA - 1.0)
    lg = jnp.log(1.0 - q)
    elem = jnp.where(ig2, 0.0, al * q * q * lg)

    aw = ax2 - ax1
    ah = ay2 - ay1
    acx = ax1 + 0.5 * aw
    acy = ay1 + 0.5 * ah
    gw0 = gx2 - gx1
    gh0 = gy2 - gy1
    gcx = gx1 + 0.5 * gw0
    gcy = gy1 + 0.5 * gh0
    gw = jnp.maximum(gw0, 1.0)
    gh = jnp.maximum(gh0, 1.0)
    dx = (gcx - acx) / aw / 0.1
    dy = (gcy - acy) / ah / 0.1
    dw = jnp.log(gw / aw) / 0.2
    dh = jnp.log(gh / ah) / 0.2
    l4 = loc_ref[0]
    loc_vec = jnp.zeros((1, _BA), jnp.float32)
    for i, d in enumerate((dx, dy, dw, dh)):
        diff = jnp.abs(d - l4[i : i + 1, :])
        sl1 = jnp.where(diff <= 1.0 / 9.0, 0.5 * 9.0 * diff * diff, diff - 0.5 / 9.0)
        loc_vec = loc_vec + jnp.where(posv, sl1, 0.0)
    loc_blk = jnp.sum(loc_vec)
    pos_cnt = jnp.sum(posv.astype(jnp.float32))
    cls_blk = jnp.sum(
        jax.lax.dot_general(
            jnp.ones((8, _K), jnp.float32),
            elem,
            (((1,), (0,)), ((), ())),
            preferred_element_type=jnp.float32,
        )[0:1, :]
    )

    @pl.when(jnp.logical_and(b == 0, j == 0))
    def _():
        acc_ref[3] = 0.0
        acc_ref[4] = 0.0

    @pl.when(j == 0)
    def _():
        acc_ref[0] = 0.0
        acc_ref[1] = 0.0
        acc_ref[2] = 0.0

    acc_ref[0] = acc_ref[0] + cls_blk
    acc_ref[1] = acc_ref[1] + loc_blk
    acc_ref[2] = acc_ref[2] + pos_cnt

    @pl.when(j == _NB - 1)
    def _():
        pn = acc_ref[2]
        cls_img = acc_ref[0] / jnp.maximum(pn, 1.0)
        loc_img = jnp.where(pn > 0.0, acc_ref[1] / jnp.maximum(pn * 4.0, 1.0), 0.0)
        acc_ref[3] = acc_ref[3] + cls_img
        acc_ref[4] = acc_ref[4] + loc_img

        @pl.when(b == _B - 1)
        def _():
            ocls_ref[0, 0] = acc_ref[3] / _B
            oloc_ref[0, 0] = acc_ref[4] / _B


def kernel(classification, localization, anchors, annotations):
    loc_t = jnp.transpose(localization, (0, 2, 1))
    anch_t = jnp.transpose(anchors)
    ocls, oloc = pl.pallas_call(
        _body,
        grid=(_B, _NB),
        in_specs=[
            pl.BlockSpec((1, _BA, _K), lambda b, j: (b, j, 0)),
            pl.BlockSpec((1, 4, _BA), lambda b, j: (b, 0, j)),
            pl.BlockSpec((4, _BA), lambda b, j: (0, j)),
            pl.BlockSpec(memory_space=pltpu.SMEM),
        ],
        out_specs=[
            pl.BlockSpec(memory_space=pltpu.SMEM),
            pl.BlockSpec(memory_space=pltpu.SMEM),
        ],
        out_shape=[
            jax.ShapeDtypeStruct((1, 1), jnp.float32),
            jax.ShapeDtypeStruct((1, 1), jnp.float32),
        ],
        scratch_shapes=[pltpu.SMEM((8,), jnp.float32)],
    )(classification, loc_t, anch_t, annotations)
    return (ocls.reshape(1), oloc.reshape(1))


# speedup vs baseline: 7.6107x; 1.0421x over previous
import jax
import jax.numpy as jnp
from jax import lax
from jax.experimental import pallas as pl
from jax.experimental.pallas import tpu as pltpu

_ALPHA = 0.25
_B, _A, _K, _M = 8, 20000, 80, 32
_BA = 5120
_NB = (_A + _BA - 1) // _BA


def _body(cls_ref, loc_ref, anch_ref, anno_ref, ocls_ref, oloc_ref, acc_ref):
    b = pl.program_id(0)
    j = pl.program_id(1)

    ax1 = anch_ref[0:1, :]
    ay1 = anch_ref[1:2, :]
    ax2 = anch_ref[2:3, :]
    ay2 = anch_ref[3:4, :]
    area_a = (ax2 - ax1) * (ay2 - ay1)

    best = jnp.full((1, _BA), -1.0, jnp.float32)
    gx1 = jnp.zeros((1, _BA), jnp.float32)
    gy1 = jnp.zeros((1, _BA), jnp.float32)
    gx2 = jnp.zeros((1, _BA), jnp.float32)
    gy2 = jnp.zeros((1, _BA), jnp.float32)
    gcl = jnp.zeros((1, _BA), jnp.float32)
    for m in range(_M):
        bx1 = anno_ref[b, m, 0]
        by1 = anno_ref[b, m, 1]
        bx2 = anno_ref[b, m, 2]
        by2 = anno_ref[b, m, 3]
        bcl = anno_ref[b, m, 4]
        area_b = (bx2 - bx1) * (by2 - by1)
        iw = jnp.maximum(jnp.minimum(ax2, bx2) - jnp.maximum(ax1, bx1), 0.0)
        ih = jnp.maximum(jnp.minimum(ay2, by2) - jnp.maximum(ay1, by1), 0.0)
        inter = iw * ih
        union = area_a + (area_b - inter)
        iou = inter / union
        cond = iou > best
        best = jnp.where(cond, iou, best)
        gx1 = jnp.where(cond, bx1, gx1)
        gy1 = jnp.where(cond, by1, gy1)
        gx2 = jnp.where(cond, bx2, gx2)
        gy2 = jnp.where(cond, by2, gy2)
        gcl = jnp.where(cond, bcl, gcl)

    gid = j * _BA + lax.broadcasted_iota(jnp.int32, (1, _BA), 1)
    valid = gid < _A
    pos = best >= 0.5
    posv = jnp.logical_and(pos, valid)
    ign = jnp.logical_or(
        jnp.logical_not(valid),
        jnp.logical_and(jnp.logical_not(pos), best >= 0.4),
    )
    code = jnp.where(ign, -1.0, jnp.where(posv, gcl, -2.0))

    p = jnp.clip(jnp.transpose(cls_ref[0], (1, 0)), 1e-4, 1.0 - 1e-4)
    kio = lax.broadcasted_iota(jnp.int32, (_K, _BA), 0).astype(jnp.float32)
    t2 = code == kio
    ig2 = code == -1.0
    om = 1.0 - p
    q = jnp.where(t2, om, p)
    r = jnp.where(t2, p, om)
    al = jnp.where(t2, -_ALPHA, _ALPHA - 1.0)
    lg = jnp.log(r)
    elem = jnp.where(ig2, 0.0, al * q * q * lg)

    aw = ax2 - ax1
    ah = ay2 - ay1
    acx = ax1 + 0.5 * aw
    acy = ay1 + 0.5 * ah
    gw0 = gx2 - gx1
    gh0 = gy2 - gy1
    gcx = gx1 + 0.5 * gw0
    gcy = gy1 + 0.5 * gh0
    gw = jnp.maximum(gw0, 1.0)
    gh = jnp.maximum(gh0, 1.0)
    dx = (gcx - acx) / aw / 0.1
    dy = (gcy - acy) / ah / 0.1
    dw = jnp.log(gw / aw) / 0.2
    dh = jnp.log(gh / ah) / 0.2
    l4 = loc_ref[0]
    loc_vec = jnp.zeros((1, _BA), jnp.float32)
    for i, d in enumerate((dx, dy, dw, dh)):
        diff = jnp.abs(d - l4[i : i + 1, :])
        sl1 = jnp.where(diff <= 1.0 / 9.0, 0.5 * 9.0 * diff * diff, diff - 0.5 / 9.0)
        loc_vec = loc_vec + jnp.where(posv, sl1, 0.0)
    loc_blk = jnp.sum(loc_vec)
    pos_cnt = jnp.sum(posv.astype(jnp.float32))
    cls_blk = jnp.sum(
        jax.lax.dot_general(
            jnp.ones((8, _K), jnp.float32),
            elem,
            (((1,), (0,)), ((), ())),
            preferred_element_type=jnp.float32,
        )[0:1, :]
    )

    @pl.when(jnp.logical_and(b == 0, j == 0))
    def _():
        acc_ref[3] = 0.0
        acc_ref[4] = 0.0

    @pl.when(j == 0)
    def _():
        acc_ref[0] = 0.0
        acc_ref[1] = 0.0
        acc_ref[2] = 0.0

    acc_ref[0] = acc_ref[0] + cls_blk
    acc_ref[1] = acc_ref[1] + loc_blk
    acc_ref[2] = acc_ref[2] + pos_cnt

    @pl.when(j == _NB - 1)
    def _():
        pn = acc_ref[2]
        cls_img = acc_ref[0] / jnp.maximum(pn, 1.0)
        loc_img = jnp.where(pn > 0.0, acc_ref[1] / jnp.maximum(pn * 4.0, 1.0), 0.0)
        acc_ref[3] = acc_ref[3] + cls_img
        acc_ref[4] = acc_ref[4] + loc_img

        @pl.when(b == _B - 1)
        def _():
            ocls_ref[0, 0] = acc_ref[3] / _B
            oloc_ref[0, 0] = acc_ref[4] / _B


def kernel(classification, localization, anchors, annotations):
    loc_t = jnp.transpose(localization, (0, 2, 1))
    anch_t = jnp.transpose(anchors)
    ocls, oloc = pl.pallas_call(
        _body,
        grid=(_B, _NB),
        in_specs=[
            pl.BlockSpec((1, _BA, _K), lambda b, j: (b, j, 0)),
            pl.BlockSpec((1, 4, _BA), lambda b, j: (b, 0, j)),
            pl.BlockSpec((4, _BA), lambda b, j: (0, j)),
            pl.BlockSpec(memory_space=pltpu.SMEM),
        ],
        out_specs=[
            pl.BlockSpec(memory_space=pltpu.SMEM),
            pl.BlockSpec(memory_space=pltpu.SMEM),
        ],
        out_shape=[
            jax.ShapeDtypeStruct((1, 1), jnp.float32),
            jax.ShapeDtypeStruct((1, 1), jnp.float32),
        ],
        scratch_shapes=[pltpu.SMEM((8,), jnp.float32)],
    )(classification, loc_t, anch_t, annotations)
    return (ocls.reshape(1), oloc.reshape(1))


# speedup vs baseline: 8.0736x; 1.0608x over previous
import jax
import jax.numpy as jnp
from jax import lax
from jax.experimental import pallas as pl
from jax.experimental.pallas import tpu as pltpu

_ALPHA = 0.25
_B, _A, _K, _M = 8, 20000, 80, 32
_BA = 10240
_NB = (_A + _BA - 1) // _BA


def _body(cls_ref, loc_ref, anch_ref, anno_ref, ocls_ref, oloc_ref, acc_ref):
    b = pl.program_id(0)
    j = pl.program_id(1)

    ax1 = anch_ref[0:1, :]
    ay1 = anch_ref[1:2, :]
    ax2 = anch_ref[2:3, :]
    ay2 = anch_ref[3:4, :]
    area_a = (ax2 - ax1) * (ay2 - ay1)

    best = jnp.full((1, _BA), -1.0, jnp.float32)
    gx1 = jnp.zeros((1, _BA), jnp.float32)
    gy1 = jnp.zeros((1, _BA), jnp.float32)
    gx2 = jnp.zeros((1, _BA), jnp.float32)
    gy2 = jnp.zeros((1, _BA), jnp.float32)
    gcl = jnp.zeros((1, _BA), jnp.float32)
    for m in range(_M):
        bx1 = anno_ref[b, m, 0]
        by1 = anno_ref[b, m, 1]
        bx2 = anno_ref[b, m, 2]
        by2 = anno_ref[b, m, 3]
        bcl = anno_ref[b, m, 4]
        area_b = (bx2 - bx1) * (by2 - by1)
        iw = jnp.maximum(jnp.minimum(ax2, bx2) - jnp.maximum(ax1, bx1), 0.0)
        ih = jnp.maximum(jnp.minimum(ay2, by2) - jnp.maximum(ay1, by1), 0.0)
        inter = iw * ih
        union = area_a + (area_b - inter)
        iou = inter / union
        cond = iou > best
        best = jnp.where(cond, iou, best)
        gx1 = jnp.where(cond, bx1, gx1)
        gy1 = jnp.where(cond, by1, gy1)
        gx2 = jnp.where(cond, bx2, gx2)
        gy2 = jnp.where(cond, by2, gy2)
        gcl = jnp.where(cond, bcl, gcl)

    gid = j * _BA + lax.broadcasted_iota(jnp.int32, (1, _BA), 1)
    valid = gid < _A
    pos = best >= 0.5
    posv = jnp.logical_and(pos, valid)
    ign = jnp.logical_or(
        jnp.logical_not(valid),
        jnp.logical_and(jnp.logical_not(pos), best >= 0.4),
    )
    code = jnp.where(ign, -1.0, jnp.where(posv, gcl, -2.0))

    p = jnp.clip(jnp.transpose(cls_ref[0], (1, 0)), 1e-4, 1.0 - 1e-4)
    kio = lax.broadcasted_iota(jnp.int32, (_K, _BA), 0).astype(jnp.float32)
    t2 = code == kio
    ig2 = code == -1.0
    om = 1.0 - p
    q = jnp.where(t2, om, p)
    r = jnp.where(t2, p, om)
    al = jnp.where(t2, -_ALPHA, _ALPHA - 1.0)
    lg = jnp.log(r)
    elem = jnp.where(ig2, 0.0, al * q * q * lg)

    aw = ax2 - ax1
    ah = ay2 - ay1
    acx = ax1 + 0.5 * aw
    acy = ay1 + 0.5 * ah
    gw0 = gx2 - gx1
    gh0 = gy2 - gy1
    gcx = gx1 + 0.5 * gw0
    gcy = gy1 + 0.5 * gh0
    gw = jnp.maximum(gw0, 1.0)
    gh = jnp.maximum(gh0, 1.0)
    dx = (gcx - acx) / aw / 0.1
    dy = (gcy - acy) / ah / 0.1
    dw = jnp.log(gw / aw) / 0.2
    dh = jnp.log(gh / ah) / 0.2
    l4 = loc_ref[0]
    loc_vec = jnp.zeros((1, _BA), jnp.float32)
    for i, d in enumerate((dx, dy, dw, dh)):
        diff = jnp.abs(d - l4[i : i + 1, :])
        sl1 = jnp.where(diff <= 1.0 / 9.0, 0.5 * 9.0 * diff * diff, diff - 0.5 / 9.0)
        loc_vec = loc_vec + jnp.where(posv, sl1, 0.0)
    loc_blk = jnp.sum(loc_vec)
    pos_cnt = jnp.sum(posv.astype(jnp.float32))
    cls_blk = jnp.sum(
        jax.lax.dot_general(
            jnp.ones((8, _K), jnp.float32),
            elem,
            (((1,), (0,)), ((), ())),
            preferred_element_type=jnp.float32,
        )[0:1, :]
    )

    @pl.when(jnp.logical_and(b == 0, j == 0))
    def _():
        acc_ref[3] = 0.0
        acc_ref[4] = 0.0

    @pl.when(j == 0)
    def _():
        acc_ref[0] = 0.0
        acc_ref[1] = 0.0
        acc_ref[2] = 0.0

    acc_ref[0] = acc_ref[0] + cls_blk
    acc_ref[1] = acc_ref[1] + loc_blk
    acc_ref[2] = acc_ref[2] + pos_cnt

    @pl.when(j == _NB - 1)
    def _():
        pn = acc_ref[2]
        cls_img = acc_ref[0] / jnp.maximum(pn, 1.0)
        loc_img = jnp.where(pn > 0.0, acc_ref[1] / jnp.maximum(pn * 4.0, 1.0), 0.0)
        acc_ref[3] = acc_ref[3] + cls_img
        acc_ref[4] = acc_ref[4] + loc_img

        @pl.when(b == _B - 1)
        def _():
            ocls_ref[0, 0] = acc_ref[3] / _B
            oloc_ref[0, 0] = acc_ref[4] / _B


def kernel(classification, localization, anchors, annotations):
    loc_t = jnp.transpose(localization, (0, 2, 1))
    anch_t = jnp.transpose(anchors)
    ocls, oloc = pl.pallas_call(
        _body,
        grid=(_B, _NB),
        in_specs=[
            pl.BlockSpec((1, _BA, _K), lambda b, j: (b, j, 0)),
            pl.BlockSpec((1, 4, _BA), lambda b, j: (b, 0, j)),
            pl.BlockSpec((4, _BA), lambda b, j: (0, j)),
            pl.BlockSpec(memory_space=pltpu.SMEM),
        ],
        out_specs=[
            pl.BlockSpec(memory_space=pltpu.SMEM),
            pl.BlockSpec(memory_space=pltpu.SMEM),
        ],
        out_shape=[
            jax.ShapeDtypeStruct((1, 1), jnp.float32),
            jax.ShapeDtypeStruct((1, 1), jnp.float32),
        ],
        scratch_shapes=[pltpu.SMEM((8,), jnp.float32)],
    )(classification, loc_t, anch_t, annotations)
    return (ocls.reshape(1), oloc.reshape(1))


# speedup vs baseline: 8.2332x; 1.0198x over previous
import jax
import jax.numpy as jnp
from jax import lax
from jax.experimental import pallas as pl
from jax.experimental.pallas import tpu as pltpu

_ALPHA = 0.25
_B, _A, _K, _M = 8, 20000, 80, 32
_BA = 20480
_NB = (_A + _BA - 1) // _BA


def _body(cls_ref, loc_ref, anch_ref, anno_ref, ocls_ref, oloc_ref, acc_ref):
    b = pl.program_id(0)
    j = pl.program_id(1)

    ax1 = anch_ref[0:1, :]
    ay1 = anch_ref[1:2, :]
    ax2 = anch_ref[2:3, :]
    ay2 = anch_ref[3:4, :]
    area_a = (ax2 - ax1) * (ay2 - ay1)

    best = jnp.full((1, _BA), -1.0, jnp.float32)
    gx1 = jnp.zeros((1, _BA), jnp.float32)
    gy1 = jnp.zeros((1, _BA), jnp.float32)
    gx2 = jnp.zeros((1, _BA), jnp.float32)
    gy2 = jnp.zeros((1, _BA), jnp.float32)
    gcl = jnp.zeros((1, _BA), jnp.float32)
    for m in range(_M):
        bx1 = anno_ref[b, m, 0]
        by1 = anno_ref[b, m, 1]
        bx2 = anno_ref[b, m, 2]
        by2 = anno_ref[b, m, 3]
        bcl = anno_ref[b, m, 4]
        area_b = (bx2 - bx1) * (by2 - by1)
        iw = jnp.maximum(jnp.minimum(ax2, bx2) - jnp.maximum(ax1, bx1), 0.0)
        ih = jnp.maximum(jnp.minimum(ay2, by2) - jnp.maximum(ay1, by1), 0.0)
        inter = iw * ih
        union = area_a + (area_b - inter)
        iou = inter / union
        cond = iou > best
        best = jnp.where(cond, iou, best)
        gx1 = jnp.where(cond, bx1, gx1)
        gy1 = jnp.where(cond, by1, gy1)
        gx2 = jnp.where(cond, bx2, gx2)
        gy2 = jnp.where(cond, by2, gy2)
        gcl = jnp.where(cond, bcl, gcl)

    gid = j * _BA + lax.broadcasted_iota(jnp.int32, (1, _BA), 1)
    valid = gid < _A
    pos = best >= 0.5
    posv = jnp.logical_and(pos, valid)
    ign = jnp.logical_or(
        jnp.logical_not(valid),
        jnp.logical_and(jnp.logical_not(pos), best >= 0.4),
    )
    code = jnp.where(ign, -1.0, jnp.where(posv, gcl, -2.0))

    p = jnp.clip(jnp.transpose(cls_ref[0], (1, 0)), 1e-4, 1.0 - 1e-4)
    kio = lax.broadcasted_iota(jnp.int32, (_K, _BA), 0).astype(jnp.float32)
    t2 = code == kio
    ig2 = code == -1.0
    om = 1.0 - p
    q = jnp.where(t2, om, p)
    r = jnp.where(t2, p, om)
    al = jnp.where(t2, -_ALPHA, _ALPHA - 1.0)
    lg = jnp.log(r)
    elem = jnp.where(ig2, 0.0, al * q * q * lg)

    aw = ax2 - ax1
    ah = ay2 - ay1
    acx = ax1 + 0.5 * aw
    acy = ay1 + 0.5 * ah
    gw0 = gx2 - gx1
    gh0 = gy2 - gy1
    gcx = gx1 + 0.5 * gw0
    gcy = gy1 + 0.5 * gh0
    gw = jnp.maximum(gw0, 1.0)
    gh = jnp.maximum(gh0, 1.0)
    dx = (gcx - acx) / aw / 0.1
    dy = (gcy - acy) / ah / 0.1
    dw = jnp.log(gw / aw) / 0.2
    dh = jnp.log(gh / ah) / 0.2
    l4 = loc_ref[0]
    loc_vec = jnp.zeros((1, _BA), jnp.float32)
    for i, d in enumerate((dx, dy, dw, dh)):
        diff = jnp.abs(d - l4[i : i + 1, :])
        sl1 = jnp.where(diff <= 1.0 / 9.0, 0.5 * 9.0 * diff * diff, diff - 0.5 / 9.0)
        loc_vec = loc_vec + jnp.where(posv, sl1, 0.0)
    loc_blk = jnp.sum(loc_vec)
    pos_cnt = jnp.sum(posv.astype(jnp.float32))
    cls_blk = jnp.sum(
        jax.lax.dot_general(
            jnp.ones((8, _K), jnp.float32),
            elem,
            (((1,), (0,)), ((), ())),
            preferred_element_type=jnp.float32,
        )[0:1, :]
    )

    @pl.when(jnp.logical_and(b == 0, j == 0))
    def _():
        acc_ref[3] = 0.0
        acc_ref[4] = 0.0

    @pl.when(j == 0)
    def _():
        acc_ref[0] = 0.0
        acc_ref[1] = 0.0
        acc_ref[2] = 0.0

    acc_ref[0] = acc_ref[0] + cls_blk
    acc_ref[1] = acc_ref[1] + loc_blk
    acc_ref[2] = acc_ref[2] + pos_cnt

    @pl.when(j == _NB - 1)
    def _():
        pn = acc_ref[2]
        cls_img = acc_ref[0] / jnp.maximum(pn, 1.0)
        loc_img = jnp.where(pn > 0.0, acc_ref[1] / jnp.maximum(pn * 4.0, 1.0), 0.0)
        acc_ref[3] = acc_ref[3] + cls_img
        acc_ref[4] = acc_ref[4] + loc_img

        @pl.when(b == _B - 1)
        def _():
            ocls_ref[0, 0] = acc_ref[3] / _B
            oloc_ref[0, 0] = acc_ref[4] / _B


def kernel(classification, localization, anchors, annotations):
    loc_t = jnp.transpose(localization, (0, 2, 1))
    anch_t = jnp.transpose(anchors)
    ocls, oloc = pl.pallas_call(
        _body,
        grid=(_B, _NB),
        in_specs=[
            pl.BlockSpec((1, _BA, _K), lambda b, j: (b, j, 0)),
            pl.BlockSpec((1, 4, _BA), lambda b, j: (b, 0, j)),
            pl.BlockSpec((4, _BA), lambda b, j: (0, j)),
            pl.BlockSpec(memory_space=pltpu.SMEM),
        ],
        out_specs=[
            pl.BlockSpec(memory_space=pltpu.SMEM),
            pl.BlockSpec(memory_space=pltpu.SMEM),
        ],
        out_shape=[
            jax.ShapeDtypeStruct((1, 1), jnp.float32),
            jax.ShapeDtypeStruct((1, 1), jnp.float32),
        ],
        scratch_shapes=[pltpu.SMEM((8,), jnp.float32)],
    )(classification, loc_t, anch_t, annotations)
    return (ocls.reshape(1), oloc.reshape(1))
